# baseline (speedup 1.0000x reference)
import jax
import jax.numpy as jnp
from jax.experimental import pallas as pl
from jax.experimental.pallas import tpu as pltpu

_IN = 768
_OUT = 768
_P = 4
_E = 8
_TILE = 1024


def _moe_kernel(x_ref, w1_ref, w2_ref, o_ref):
    x = x_ref[...]
    ht = jax.lax.dot_general(
        w1_ref[...], x,
        dimension_numbers=(((0,), (1,)), ((), ())),
        preferred_element_type=jnp.float32,
    )
    h3 = ht.reshape(_P, _E, x.shape[0])
    m = jnp.max(h3, axis=1, keepdims=True)
    eidx = jax.lax.broadcasted_iota(jnp.int32, h3.shape, 1)
    first = jnp.min(jnp.where(h3 == m, eidx, _E), axis=1, keepdims=True)
    z3 = jnp.where(eidx == first, h3, 0.0)
    z = z3.reshape(_P * _E, x.shape[0])
    o_ref[...] = jax.lax.dot_general(
        z, w2_ref[...],
        dimension_numbers=(((0,), (0,)), ((), ())),
        preferred_element_type=jnp.float32,
    )


def kernel(x, w1, w2):
    s = x.shape
    xf = x.reshape(-1, _IN)
    t = xf.shape[0]
    out = pl.pallas_call(
        _moe_kernel,
        grid=(t // _TILE,),
        in_specs=[
            pl.BlockSpec((_TILE, _IN), lambda i: (i, 0)),
            pl.BlockSpec((_IN, _P * _E), lambda i: (0, 0)),
            pl.BlockSpec((_P * _E, _OUT), lambda i: (0, 0)),
        ],
        out_specs=pl.BlockSpec((_TILE, _OUT), lambda i: (i, 0)),
        out_shape=jax.ShapeDtypeStruct((t, _OUT), jnp.float32),
        compiler_params=pltpu.CompilerParams(
            dimension_semantics=("arbitrary",),
        ),
    )(xf, w1.reshape(_IN, _P * _E), w2.reshape(_P * _E, _OUT))
    return out.reshape(s[:-1] + (_OUT,))


# speedup vs baseline: 1.1075x; 1.1075x over previous
import jax
import jax.numpy as jnp
from jax.experimental import pallas as pl
from jax.experimental.pallas import tpu as pltpu

_IN = 768
_OUT = 768
_P = 4
_E = 8
_TILE = 4096


def _moe_kernel(x_ref, w1_ref, w2_ref, o_ref):
    x = x_ref[...]
    ht = jax.lax.dot_general(
        w1_ref[...], x,
        dimension_numbers=(((0,), (1,)), ((), ())),
        preferred_element_type=jnp.float32,
    )
    h3 = ht.reshape(_P, _E, x.shape[0])
    m = jnp.max(h3, axis=1, keepdims=True)
    eidx = jax.lax.broadcasted_iota(jnp.int32, h3.shape, 1)
    first = jnp.min(jnp.where(h3 == m, eidx, _E), axis=1, keepdims=True)
    z3 = jnp.where(eidx == first, h3, 0.0)
    z = z3.reshape(_P * _E, x.shape[0])
    o_ref[...] = jax.lax.dot_general(
        z, w2_ref[...],
        dimension_numbers=(((0,), (0,)), ((), ())),
        preferred_element_type=jnp.float32,
    )


def kernel(x, w1, w2):
    s = x.shape
    xf = x.reshape(-1, _IN)
    t = xf.shape[0]
    out = pl.pallas_call(
        _moe_kernel,
        grid=(t // _TILE,),
        in_specs=[
            pl.BlockSpec((_TILE, _IN), lambda i: (i, 0)),
            pl.BlockSpec((_IN, _P * _E), lambda i: (0, 0)),
            pl.BlockSpec((_P * _E, _OUT), lambda i: (0, 0)),
        ],
        out_specs=pl.BlockSpec((_TILE, _OUT), lambda i: (i, 0)),
        out_shape=jax.ShapeDtypeStruct((t, _OUT), jnp.float32),
        compiler_params=pltpu.CompilerParams(
            dimension_semantics=("arbitrary",),
        ),
    )(xf, w1.reshape(_IN, _P * _E), w2.reshape(_P * _E, _OUT))
    return out.reshape(s[:-1] + (_OUT,))
